# Initial kernel scaffold; baseline (speedup 1.0000x reference)
import jax
import jax.numpy as jnp
from jax.experimental import pallas as pl


def kernel(x, edge_index, params):
    raise NotImplementedError("write your pallas kernel here")


# speedup vs baseline: 7.2923x; 7.2923x over previous
import functools

import jax
import jax.numpy as jnp
from jax import lax
from jax.experimental import pallas as pl
from jax.experimental.pallas import tpu as pltpu
from jax.experimental.pallas import tpu_sc as plsc

N_NODES = 10000
N_EDGES = 320000
HID = 128
NC = 2
NS = 16
N_TILES = NC * NS
EDGES_PER_TILE = N_EDGES // N_TILES
CHUNK = 80
N_CHUNKS = EDGES_PER_TILE // CHUNK
ROWS_A = 624
ROWS_LAST = N_NODES - (NS - 1) * ROWS_A
BN_INV_STD = 1.0 / (1.0 + 1e-5) ** 0.5


def _sc_aggregate(h, src, dst, zeros):
  mesh = plsc.VectorSubcoreMesh(core_axis_name="c", subcore_axis_name="s")

  @functools.partial(
      pl.kernel,
      out_type=jax.ShapeDtypeStruct((NC, N_NODES, HID), jnp.float32),
      mesh=mesh,
      scratch_types=[
          pltpu.VMEM((N_CHUNKS, CHUNK), jnp.int32),
          pltpu.VMEM((N_CHUNKS, CHUNK), jnp.int32),
          pltpu.VMEM((CHUNK, HID), jnp.float32),
          pltpu.VMEM_SHARED((N_NODES, HID), jnp.float32),
          pltpu.SemaphoreType.DMA,
      ],
  )
  def k(h_hbm, src_hbm, dst_hbm, z_hbm, out_hbm, sidx, didx, rows, acc, sem):
    c = lax.axis_index("c")
    s = lax.axis_index("s")
    wid = c * NS + s
    row0 = s * ROWS_A
    is_last = s == NS - 1

    @pl.when(jnp.logical_and(c == 0, jnp.logical_not(is_last)))
    def _():
      pltpu.sync_copy(h_hbm.at[pl.ds(row0, ROWS_A)],
                      acc.at[pl.ds(row0, ROWS_A)])

    @pl.when(jnp.logical_and(c == 0, is_last))
    def _():
      pltpu.sync_copy(h_hbm.at[pl.ds(row0, ROWS_LAST)],
                      acc.at[pl.ds(row0, ROWS_LAST)])

    @pl.when(jnp.logical_and(c != 0, jnp.logical_not(is_last)))
    def _():
      pltpu.sync_copy(z_hbm.at[pl.ds(0, ROWS_A)], acc.at[pl.ds(row0, ROWS_A)])

    @pl.when(jnp.logical_and(c != 0, is_last))
    def _():
      pltpu.sync_copy(z_hbm, acc.at[pl.ds(row0, ROWS_LAST)])

    pltpu.sync_copy(src_hbm.at[wid], sidx)
    pltpu.sync_copy(dst_hbm.at[wid], didx)
    plsc.subcore_barrier()

    @pl.loop(0, N_CHUNKS)
    def _(j):
      pltpu.async_copy(h_hbm.at[sidx.at[j]], rows, sem).wait()
      pltpu.sync_copy(rows, acc.at[didx.at[j]], add=True)

    plsc.subcore_barrier()

    @pl.when(jnp.logical_not(is_last))
    def _():
      pltpu.sync_copy(acc.at[pl.ds(row0, ROWS_A)],
                      out_hbm.at[c, pl.ds(row0, ROWS_A)])

    @pl.when(is_last)
    def _():
      pltpu.sync_copy(acc.at[pl.ds(row0, ROWS_LAST)],
                      out_hbm.at[c, pl.ds(row0, ROWS_LAST)])

  return k(h, src, dst, zeros)


def _mlp_mid(parts, W1, b1, W2, b2, scale, beta):
  BLK = 2000

  def body(p_ref, w1_ref, b1_ref, w2_ref, b2_ref, sc_ref, be_ref, o_ref):
    m = p_ref[0] + p_ref[1]
    t = jnp.dot(m, w1_ref[...], preferred_element_type=jnp.float32)
    t = jnp.maximum(t + b1_ref[...], 0.0)
    u = jnp.dot(t, w2_ref[...], preferred_element_type=jnp.float32)
    u = (u + b2_ref[...]) * sc_ref[...] + be_ref[...]
    o_ref[...] = jnp.maximum(u, 0.0)

  return pl.pallas_call(
      body,
      grid=(N_NODES // BLK,),
      in_specs=[
          pl.BlockSpec((NC, BLK, HID), lambda i: (0, i, 0)),
          pl.BlockSpec((HID, HID), lambda i: (0, 0)),
          pl.BlockSpec((1, HID), lambda i: (0, 0)),
          pl.BlockSpec((HID, HID), lambda i: (0, 0)),
          pl.BlockSpec((1, HID), lambda i: (0, 0)),
          pl.BlockSpec((1, HID), lambda i: (0, 0)),
          pl.BlockSpec((1, HID), lambda i: (0, 0)),
      ],
      out_specs=pl.BlockSpec((BLK, HID), lambda i: (i, 0)),
      out_shape=jax.ShapeDtypeStruct((N_NODES, HID), jnp.float32),
  )(parts, W1, b1, W2, b2, scale, beta)


def _mlp_last(parts, W1, b1, W2, b2, scale, beta):
  BLK = 2000

  def body(p_ref, w1_ref, b1_ref, w2_ref, b2_ref, sc_ref, be_ref, o_ref):
    m = p_ref[0] + p_ref[1]
    t = jnp.dot(m, w1_ref[...], preferred_element_type=jnp.float32)
    t = jnp.maximum(t + b1_ref[...], 0.0)
    u = jnp.dot(t, w2_ref[...], preferred_element_type=jnp.float32)
    u = (u + b2_ref[...]) * sc_ref[...] + be_ref[...]
    h_blk = jnp.maximum(u, 0.0)
    part = jnp.sum(h_blk, axis=0, keepdims=True) * (1.0 / N_NODES)

    @pl.when(pl.program_id(0) == 0)
    def _():
      o_ref[...] = part

    @pl.when(pl.program_id(0) != 0)
    def _():
      o_ref[...] += part

  return pl.pallas_call(
      body,
      grid=(N_NODES // BLK,),
      in_specs=[
          pl.BlockSpec((NC, BLK, HID), lambda i: (0, i, 0)),
          pl.BlockSpec((HID, HID), lambda i: (0, 0)),
          pl.BlockSpec((1, HID), lambda i: (0, 0)),
          pl.BlockSpec((HID, HID), lambda i: (0, 0)),
          pl.BlockSpec((1, HID), lambda i: (0, 0)),
          pl.BlockSpec((1, HID), lambda i: (0, 0)),
          pl.BlockSpec((1, HID), lambda i: (0, 0)),
      ],
      out_specs=pl.BlockSpec((1, HID), lambda i: (0, 0)),
      out_shape=jax.ShapeDtypeStruct((1, HID), jnp.float32),
  )(parts, W1, b1, W2, b2, scale, beta)


def kernel(x, edge_index, params):
  src = edge_index[0].reshape(N_TILES, N_CHUNKS, CHUNK)
  dst = edge_index[1].reshape(N_TILES, N_CHUNKS, CHUNK)
  zeros = jnp.zeros((ROWS_LAST, HID), jnp.float32)
  h = x.astype(jnp.float32)
  n_layers = len(params)
  for i, (W1, b1, W2, b2, gamma, beta) in enumerate(params):
    parts = _sc_aggregate(h, src, dst, zeros)
    scale = (gamma * BN_INV_STD).reshape(1, HID)
    args = (parts, W1, b1.reshape(1, HID), W2, b2.reshape(1, HID),
            scale, beta.reshape(1, HID))
    if i == n_layers - 1:
      h = _mlp_last(*args)
    else:
      h = _mlp_mid(*args)
  return h


# speedup vs baseline: 13.1601x; 1.8047x over previous
import functools

import jax
import jax.numpy as jnp
from jax import lax
from jax.experimental import pallas as pl
from jax.experimental.pallas import tpu as pltpu
from jax.experimental.pallas import tpu_sc as plsc

N_NODES = 10000
N_EDGES = 320000
HID = 128
NC = 2
NS = 16
N_TILES = NC * NS
EDGES_PER_TILE = N_EDGES // N_TILES
CHUNK = 125
N_CHUNKS = EDGES_PER_TILE // CHUNK
STAGE = 16
N_STAGES = N_CHUNKS // STAGE
ROWS_A = 624
ROWS_LAST = N_NODES - (NS - 1) * ROWS_A
BN_INV_STD = 1.0 / (1.0 + 1e-5) ** 0.5


def _sc_aggregate(h, src, dst, zeros):
  mesh = plsc.VectorSubcoreMesh(core_axis_name="c", subcore_axis_name="s")

  @functools.partial(
      pl.kernel,
      out_type=jax.ShapeDtypeStruct((NC, N_NODES, HID), jnp.float32),
      mesh=mesh,
      scratch_types=[
          pltpu.VMEM((N_CHUNKS, CHUNK), jnp.int32),
          pltpu.VMEM((STAGE, CHUNK), jnp.int32),
          pltpu.VMEM((STAGE, CHUNK), jnp.int32),
          pltpu.VMEM((CHUNK, HID), jnp.float32),
          pltpu.VMEM((CHUNK, HID), jnp.float32),
          pltpu.VMEM_SHARED((N_NODES, HID), jnp.float32),
          pltpu.SemaphoreType.DMA,
          pltpu.SemaphoreType.DMA,
          pltpu.SemaphoreType.DMA,
          pltpu.SemaphoreType.DMA,
      ],
  )
  def k(h_hbm, src_hbm, dst_hbm, z_hbm, out_hbm,
        sidx, didx0, didx1, rows0, rows1, acc, sem0, sem1, dsem0, dsem1):
    c = lax.axis_index("c")
    s = lax.axis_index("s")
    wid = c * NS + s
    row0 = s * ROWS_A
    is_last = s == NS - 1

    @pl.when(jnp.logical_and(c == 0, jnp.logical_not(is_last)))
    def _():
      pltpu.sync_copy(h_hbm.at[pl.ds(row0, ROWS_A)],
                      acc.at[pl.ds(row0, ROWS_A)])

    @pl.when(jnp.logical_and(c == 0, is_last))
    def _():
      pltpu.sync_copy(h_hbm.at[pl.ds(row0, ROWS_LAST)],
                      acc.at[pl.ds(row0, ROWS_LAST)])

    @pl.when(jnp.logical_and(c != 0, jnp.logical_not(is_last)))
    def _():
      pltpu.sync_copy(z_hbm.at[pl.ds(0, ROWS_A)], acc.at[pl.ds(row0, ROWS_A)])

    @pl.when(jnp.logical_and(c != 0, is_last))
    def _():
      pltpu.sync_copy(z_hbm, acc.at[pl.ds(row0, ROWS_LAST)])

    pltpu.sync_copy(src_hbm.at[wid], sidx)
    dbufs = (didx0, didx1)
    dsems = (dsem0, dsem1)
    dcps = [pltpu.async_copy(dst_hbm.at[wid, pl.ds(0, STAGE)], didx0, dsem0)]
    plsc.subcore_barrier()

    cp0 = pltpu.async_copy(h_hbm.at[sidx.at[0]], rows0, sem0)
    cp1 = pltpu.async_copy(h_hbm.at[sidx.at[1]], rows1, sem1)

    for st in range(N_STAGES):
      if st + 1 < N_STAGES:
        dcps.append(pltpu.async_copy(
            dst_hbm.at[wid, pl.ds((st + 1) * STAGE, STAGE)],
            dbufs[(st + 1) % 2], dsems[(st + 1) % 2]))
      dcps[st].wait()
      dbuf = dbufs[st % 2]
      base = st * STAGE

      @pl.loop(0, STAGE, step=2)
      def _(jj):
        j = base + jj
        cp0.wait()
        pltpu.sync_copy(rows0, acc.at[dbuf.at[jj]], add=True)

        @pl.when(j + 2 < N_CHUNKS)
        def _():
          pltpu.async_copy(h_hbm.at[sidx.at[j + 2]], rows0, sem0)

        cp1.wait()
        pltpu.sync_copy(rows1, acc.at[dbuf.at[jj + 1]], add=True)

        @pl.when(j + 3 < N_CHUNKS)
        def _():
          pltpu.async_copy(h_hbm.at[sidx.at[j + 3]], rows1, sem1)

    plsc.subcore_barrier()

    @pl.when(jnp.logical_not(is_last))
    def _():
      pltpu.sync_copy(acc.at[pl.ds(row0, ROWS_A)],
                      out_hbm.at[c, pl.ds(row0, ROWS_A)])

    @pl.when(is_last)
    def _():
      pltpu.sync_copy(acc.at[pl.ds(row0, ROWS_LAST)],
                      out_hbm.at[c, pl.ds(row0, ROWS_LAST)])

  return k(h, src, dst, zeros)


def _mlp_mid(parts, W1, b1, W2, b2, scale, beta):
  BLK = 2000

  def body(p_ref, w1_ref, b1_ref, w2_ref, b2_ref, sc_ref, be_ref, o_ref):
    m = p_ref[0] + p_ref[1]
    t = jnp.dot(m, w1_ref[...], preferred_element_type=jnp.float32)
    t = jnp.maximum(t + b1_ref[...], 0.0)
    u = jnp.dot(t, w2_ref[...], preferred_element_type=jnp.float32)
    u = (u + b2_ref[...]) * sc_ref[...] + be_ref[...]
    o_ref[...] = jnp.maximum(u, 0.0)

  return pl.pallas_call(
      body,
      grid=(N_NODES // BLK,),
      in_specs=[
          pl.BlockSpec((NC, BLK, HID), lambda i: (0, i, 0)),
          pl.BlockSpec((HID, HID), lambda i: (0, 0)),
          pl.BlockSpec((1, HID), lambda i: (0, 0)),
          pl.BlockSpec((HID, HID), lambda i: (0, 0)),
          pl.BlockSpec((1, HID), lambda i: (0, 0)),
          pl.BlockSpec((1, HID), lambda i: (0, 0)),
          pl.BlockSpec((1, HID), lambda i: (0, 0)),
      ],
      out_specs=pl.BlockSpec((BLK, HID), lambda i: (i, 0)),
      out_shape=jax.ShapeDtypeStruct((N_NODES, HID), jnp.float32),
  )(parts, W1, b1, W2, b2, scale, beta)


def _mlp_last(parts, W1, b1, W2, b2, scale, beta):
  BLK = 2000

  def body(p_ref, w1_ref, b1_ref, w2_ref, b2_ref, sc_ref, be_ref, o_ref):
    m = p_ref[0] + p_ref[1]
    t = jnp.dot(m, w1_ref[...], preferred_element_type=jnp.float32)
    t = jnp.maximum(t + b1_ref[...], 0.0)
    u = jnp.dot(t, w2_ref[...], preferred_element_type=jnp.float32)
    u = (u + b2_ref[...]) * sc_ref[...] + be_ref[...]
    h_blk = jnp.maximum(u, 0.0)
    part = jnp.sum(h_blk, axis=0, keepdims=True) * (1.0 / N_NODES)

    @pl.when(pl.program_id(0) == 0)
    def _():
      o_ref[...] = part

    @pl.when(pl.program_id(0) != 0)
    def _():
      o_ref[...] += part

  return pl.pallas_call(
      body,
      grid=(N_NODES // BLK,),
      in_specs=[
          pl.BlockSpec((NC, BLK, HID), lambda i: (0, i, 0)),
          pl.BlockSpec((HID, HID), lambda i: (0, 0)),
          pl.BlockSpec((1, HID), lambda i: (0, 0)),
          pl.BlockSpec((HID, HID), lambda i: (0, 0)),
          pl.BlockSpec((1, HID), lambda i: (0, 0)),
          pl.BlockSpec((1, HID), lambda i: (0, 0)),
          pl.BlockSpec((1, HID), lambda i: (0, 0)),
      ],
      out_specs=pl.BlockSpec((1, HID), lambda i: (0, 0)),
      out_shape=jax.ShapeDtypeStruct((1, HID), jnp.float32),
  )(parts, W1, b1, W2, b2, scale, beta)


def kernel(x, edge_index, params):
  src = edge_index[0].reshape(N_TILES, N_CHUNKS, CHUNK)
  dst = edge_index[1].reshape(N_TILES, N_CHUNKS, CHUNK)
  zeros = jnp.zeros((ROWS_LAST, HID), jnp.float32)
  h = x.astype(jnp.float32)
  n_layers = len(params)
  for i, (W1, b1, W2, b2, gamma, beta) in enumerate(params):
    parts = _sc_aggregate(h, src, dst, zeros)
    scale = (gamma * BN_INV_STD).reshape(1, HID)
    args = (parts, W1, b1.reshape(1, HID), W2, b2.reshape(1, HID),
            scale, beta.reshape(1, HID))
    if i == n_layers - 1:
      h = _mlp_last(*args)
    else:
      h = _mlp_mid(*args)
  return h
